# baseline (speedup 1.0000x reference)
import functools

import jax
import jax.numpy as jnp
from jax import lax
from jax.experimental import pallas as pl
from jax.experimental.pallas import tpu as pltpu
from jax.experimental.pallas import tpu_sc as plsc

B = 512
NUM_AGENTS = 64
NUM_HIVES = 16
NUM_ENTITIES = 80
LANES = 16
NUM_WORKERS = 32
B_PER_W = B // NUM_WORKERS


def _sc_vote_body(votes_hbm, hv_hbm, mf_hbm, part_hbm,
                  votes_v, hv_v, counts_v, mf_v, part_v):
    c = lax.axis_index("c")
    s = lax.axis_index("s")
    wid = s * 2 + c
    b0 = wid * B_PER_W

    row = NUM_AGENTS * NUM_HIVES
    pltpu.sync_copy(votes_hbm.at[pl.ds(b0 * row, B_PER_W * row)], votes_v)
    pltpu.sync_copy(hv_hbm.at[pl.ds(b0 * NUM_HIVES, B_PER_W * NUM_HIVES)], hv_v)

    lane = lax.iota(jnp.int32, LANES)
    lane_row = lane * row
    lane_hv = lane * NUM_HIVES
    zero16 = jnp.zeros((LANES,), jnp.float32)
    ones16 = jnp.ones((LANES,), jnp.float32)
    for h in range(NUM_HIVES):
        counts_v[pl.ds(h * LANES, LANES)] = zero16

    UNROLL = 8

    def agent_body(i, carry):
        a0 = i * UNROLL
        for u in range(UNROLL):
            base = lane_row + (a0 + u) * NUM_HIVES
            best_val = jnp.full((LANES,), -jnp.inf, jnp.float32)
            best_idx = jnp.zeros((LANES,), jnp.int32)
            for h in range(NUM_HIVES):
                h_splat = jnp.full((LANES,), h, jnp.int32)
                col = plsc.load_gather(votes_v, [base + h])
                m = col > best_val
                best_val = jnp.where(m, col, best_val)
                best_idx = jnp.where(m, h_splat, best_idx)
            plsc.addupdate_scatter(counts_v, [best_idx * LANES + lane], ones16)
        return carry

    lax.fori_loop(0, NUM_AGENTS // UNROLL, agent_body, 0)

    mf = zero16
    val = zero16
    for h in range(NUM_HIVES):
        ch = counts_v[pl.ds(h * LANES, LANES)]
        mf = jnp.maximum(mf, ch)
        hcol = plsc.load_gather(hv_v, [lane_hv + h])
        val = val + ch * hcol

    max_freq = mf * (1.0 / NUM_AGENTS)
    x = 30.0 * (max_freq - 0.7)
    term = val * (1.0 + jnp.exp(x)) * (1.0 / 100.0)

    mf_v[...] = max_freq
    part_v[...] = term
    pltpu.sync_copy(mf_v, mf_hbm.at[pl.ds(b0, B_PER_W)])
    pltpu.sync_copy(part_v, part_hbm.at[wid])


_sc_vote = functools.partial(
    pl.kernel,
    out_type=(jax.ShapeDtypeStruct((B,), jnp.float32),
              jax.ShapeDtypeStruct((NUM_WORKERS, LANES), jnp.float32)),
    mesh=plsc.VectorSubcoreMesh(core_axis_name="c", subcore_axis_name="s"),
    compiler_params=pltpu.CompilerParams(needs_layout_passes=False),
    scratch_types=[
        pltpu.VMEM((B_PER_W * NUM_AGENTS * NUM_HIVES,), jnp.float32),
        pltpu.VMEM((B_PER_W * NUM_HIVES,), jnp.float32),
        pltpu.VMEM((NUM_HIVES * LANES,), jnp.float32),
        pltpu.VMEM((LANES,), jnp.float32),
        pltpu.VMEM((LANES,), jnp.float32),
    ],
)(_sc_vote_body)


def _tc_cost_body(mv_ref, part_ref, out_ref):
    a = mv_ref[...]
    sq = a * a
    i = lax.broadcasted_iota(jnp.int32, (2 * NUM_ENTITIES, NUM_ENTITIES), 0)
    j = lax.broadcasted_iota(jnp.int32, (2 * NUM_ENTITIES, NUM_ENTITIES), 1)
    pair = (i // 2 == j).astype(jnp.float32)
    s2 = jnp.dot(sq, pair, preferred_element_type=jnp.float32)
    total = jnp.sum(jnp.sqrt(s2)) - jnp.sum(part_ref[...])
    out_ref[...] = jnp.reshape(total, (1, 1))


def kernel(movements, utterances, votes, hive_values, locations):
    hv = hive_values.reshape(B * NUM_HIVES)
    max_freq, parts = _sc_vote(votes.reshape(B * NUM_AGENTS * NUM_HIVES), hv)
    mv2d = movements.reshape(B, 2 * NUM_ENTITIES)
    cost = pl.pallas_call(
        _tc_cost_body,
        out_shape=jax.ShapeDtypeStruct((1, 1), jnp.float32),
    )(mv2d, parts)
    return (cost[0, 0], max_freq)


# speedup vs baseline: 1.8963x; 1.8963x over previous
import functools

import jax
import jax.numpy as jnp
from jax import lax
from jax.experimental import pallas as pl
from jax.experimental.pallas import tpu as pltpu
from jax.experimental.pallas import tpu_sc as plsc

B = 512
NUM_AGENTS = 64
NUM_HIVES = 16
NUM_ENTITIES = 80
LANES = 16
NUM_WORKERS = 32
A_PER_W = NUM_AGENTS // NUM_WORKERS
NUM_BG = B // LANES


def _sc_vote_body(votes_hbm, cnt_hbm, votes_v, counts_v):
    c = lax.axis_index("c")
    s = lax.axis_index("s")
    wid = s * 2 + c
    a0 = wid * A_PER_W

    pltpu.sync_copy(votes_hbm.at[pl.ds(a0, A_PER_W)], votes_v)

    lane = lax.iota(jnp.int32, LANES)
    zero16 = jnp.zeros((LANES,), jnp.float32)
    ones16 = jnp.ones((LANES,), jnp.float32)

    def zero_body(j, carry):
        for k in range(16):
            counts_v[pl.ds(j * 256 + k * LANES, LANES)] = zero16
        return carry

    lax.fori_loop(0, NUM_HIVES * B // (16 * LANES), zero_body, 0)

    def bg_body(bg, carry):
        b_vec = bg * LANES + lane
        for a in range(A_PER_W):
            best_val = jnp.full((LANES,), -jnp.inf, jnp.float32)
            best_idx = jnp.zeros((LANES,), jnp.int32)
            for h in range(NUM_HIVES):
                col = votes_v[a, h, pl.ds(bg * LANES, LANES)]
                m = col > best_val
                best_val = jnp.where(m, col, best_val)
                best_idx = jnp.where(m, jnp.full((LANES,), h, jnp.int32), best_idx)
            plsc.addupdate_scatter(counts_v, [best_idx * B + b_vec], ones16)
        return carry

    lax.fori_loop(0, NUM_BG, bg_body, 0)

    pltpu.sync_copy(counts_v, cnt_hbm.at[wid])


_sc_vote = functools.partial(
    pl.kernel,
    out_type=jax.ShapeDtypeStruct((NUM_WORKERS, NUM_HIVES * B), jnp.float32),
    mesh=plsc.VectorSubcoreMesh(core_axis_name="c", subcore_axis_name="s"),
    compiler_params=pltpu.CompilerParams(needs_layout_passes=False),
    scratch_types=[
        pltpu.VMEM((A_PER_W, NUM_HIVES, B), jnp.float32),
        pltpu.VMEM((NUM_HIVES * B,), jnp.float32),
    ],
)(_sc_vote_body)


def _tc_cost_body(cnt_ref, hv_ref, mv_ref, cost_ref, mf_ref):
    counts = jnp.sum(cnt_ref[...], axis=0)
    mf = jnp.max(counts, axis=0) * (1.0 / NUM_AGENTS)
    val = jnp.sum(counts * hv_ref[...], axis=0)
    term = val * (1.0 + jnp.exp(30.0 * (mf - 0.7))) * (1.0 / 100.0)

    mv = mv_ref[...]
    norms = jnp.sqrt(jnp.sum(mv * mv, axis=1))

    cost_ref[...] = jnp.reshape(jnp.sum(norms) - jnp.sum(term), (1, 1))
    mf_ref[...] = jnp.reshape(mf, (1, B))


def kernel(movements, utterances, votes, hive_values, locations):
    votes_t = jnp.transpose(votes, (1, 2, 0))
    parts = _sc_vote(votes_t)
    parts3 = parts.reshape(NUM_WORKERS, NUM_HIVES, B)
    hv_t = jnp.transpose(jnp.squeeze(hive_values, -1))
    mv_t = jnp.transpose(movements, (1, 2, 0))
    cost, mf = pl.pallas_call(
        _tc_cost_body,
        out_shape=(jax.ShapeDtypeStruct((1, 1), jnp.float32),
                   jax.ShapeDtypeStruct((1, B), jnp.float32)),
    )(parts3, hv_t, mv_t)
    return (cost[0, 0], mf.reshape(B))


# speedup vs baseline: 2.0691x; 1.0911x over previous
import functools

import jax
import jax.numpy as jnp
from jax import lax
from jax.experimental import pallas as pl
from jax.experimental.pallas import tpu as pltpu
from jax.experimental.pallas import tpu_sc as plsc

B = 512
NUM_AGENTS = 64
NUM_HIVES = 16
NUM_ENTITIES = 80
LANES = 16
NUM_WORKERS = 32
A_PER_W = NUM_AGENTS // NUM_WORKERS
NUM_BG = B // LANES


def _sc_vote_body(votes_hbm, cnt_hbm, votes_v, counts_v):
    c = lax.axis_index("c")
    s = lax.axis_index("s")
    wid = s * 2 + c
    a0 = wid * A_PER_W

    pltpu.sync_copy(votes_hbm.at[pl.ds(a0, A_PER_W)], votes_v)

    lane = lax.iota(jnp.int32, LANES)
    zero16 = jnp.zeros((LANES,), jnp.float32)
    ones16 = jnp.ones((LANES,), jnp.float32)

    def zero_body(j, carry):
        for k in range(16):
            counts_v[pl.ds(j * 256 + k * LANES, LANES)] = zero16
        return carry

    lax.fori_loop(0, NUM_HIVES * B // (16 * LANES), zero_body, 0)

    def half_scan(a, bg, h_lo):
        best_val = votes_v[a, h_lo, pl.ds(bg * LANES, LANES)]
        best_idx = jnp.full((LANES,), h_lo, jnp.int32)
        for h in range(h_lo + 1, h_lo + 8):
            col = votes_v[a, h, pl.ds(bg * LANES, LANES)]
            m = col > best_val
            best_val = jnp.where(m, col, best_val)
            best_idx = jnp.where(m, jnp.full((LANES,), h, jnp.int32), best_idx)
        return best_val, best_idx

    def bg_body(bg, carry):
        b_vec = bg * LANES + lane
        for a in range(A_PER_W):
            val_lo, idx_lo = half_scan(a, bg, 0)
            val_hi, idx_hi = half_scan(a, bg, 8)
            m = val_hi > val_lo
            best_idx = jnp.where(m, idx_hi, idx_lo)
            plsc.addupdate_scatter(counts_v, [best_idx * B + b_vec], ones16)
        return carry

    lax.fori_loop(0, NUM_BG, bg_body, 0)

    for h in range(NUM_HIVES):
        pltpu.sync_copy(counts_v.at[pl.ds(h * B, B)],
                        cnt_hbm.at[wid * NUM_HIVES + h])


_sc_vote = functools.partial(
    pl.kernel,
    out_type=jax.ShapeDtypeStruct((NUM_WORKERS * NUM_HIVES, B), jnp.float32),
    mesh=plsc.VectorSubcoreMesh(core_axis_name="c", subcore_axis_name="s"),
    compiler_params=pltpu.CompilerParams(needs_layout_passes=False),
    scratch_types=[
        pltpu.VMEM((A_PER_W, NUM_HIVES, B), jnp.float32),
        pltpu.VMEM((NUM_HIVES * B,), jnp.float32),
    ],
)(_sc_vote_body)


def _tc_cost_body(cnt_ref, hv_ref, mv_ref, cost_ref, mf_ref):
    parts = cnt_ref[...].reshape(NUM_WORKERS, NUM_HIVES, B)
    counts = jnp.sum(parts, axis=0)
    mf = jnp.max(counts, axis=0) * (1.0 / NUM_AGENTS)
    val = jnp.sum(counts * hv_ref[...], axis=0)
    term = val * (1.0 + jnp.exp(30.0 * (mf - 0.7))) * (1.0 / 100.0)

    mv = mv_ref[...]
    norms = jnp.sqrt(jnp.sum(mv * mv, axis=1))

    cost_ref[...] = jnp.reshape(jnp.sum(norms) - jnp.sum(term), (1, 1))
    mf_ref[...] = jnp.reshape(mf, (1, B))


def kernel(movements, utterances, votes, hive_values, locations):
    votes_t = jnp.transpose(votes, (1, 2, 0))
    parts = _sc_vote(votes_t)
    hv_t = jnp.transpose(jnp.squeeze(hive_values, -1))
    mv_t = jnp.transpose(movements, (1, 2, 0))
    cost, mf = pl.pallas_call(
        _tc_cost_body,
        out_shape=(jax.ShapeDtypeStruct((1, 1), jnp.float32),
                   jax.ShapeDtypeStruct((1, B), jnp.float32)),
    )(parts, hv_t, mv_t)
    return (cost[0, 0], mf.reshape(B))


# speedup vs baseline: 2.1422x; 1.0353x over previous
import functools

import jax
import jax.numpy as jnp
from jax import lax
from jax.experimental import pallas as pl
from jax.experimental.pallas import tpu as pltpu
from jax.experimental.pallas import tpu_sc as plsc

B = 512
NUM_AGENTS = 64
NUM_HIVES = 16
NUM_ENTITIES = 80
LANES = 16
NUM_WORKERS = 32
A_PER_W = NUM_AGENTS // NUM_WORKERS
NUM_BG = B // LANES


def _sc_vote_body(votes_hbm, cnt_hbm, votes_v, counts_v):
    c = lax.axis_index("c")
    s = lax.axis_index("s")
    wid = s * 2 + c
    a0 = wid * A_PER_W

    pltpu.sync_copy(votes_hbm.at[pl.ds(a0, A_PER_W)], votes_v)

    lane = lax.iota(jnp.int32, LANES)
    zero16 = jnp.zeros((LANES,), jnp.float32)
    ones16 = jnp.ones((LANES,), jnp.float32)

    def zero_body(j, carry):
        for h in range(NUM_HIVES):
            counts_v[h, pl.ds(j * LANES, LANES)] = zero16
        return carry

    lax.fori_loop(0, B // LANES, zero_body, 0)

    def half_scan(a, bg, h_lo):
        best_val = votes_v[a, h_lo, pl.ds(bg * LANES, LANES)]
        best_idx = jnp.full((LANES,), h_lo, jnp.int32)
        for h in range(h_lo + 1, h_lo + 8):
            col = votes_v[a, h, pl.ds(bg * LANES, LANES)]
            m = col > best_val
            best_val = jnp.where(m, col, best_val)
            best_idx = jnp.where(m, jnp.full((LANES,), h, jnp.int32), best_idx)
        return best_val, best_idx

    def bg_body(bg, carry):
        b_vec = bg * LANES + lane
        for a in range(A_PER_W):
            val_lo, idx_lo = half_scan(a, bg, 0)
            val_hi, idx_hi = half_scan(a, bg, 8)
            m = val_hi > val_lo
            best_idx = jnp.where(m, idx_hi, idx_lo)
            plsc.addupdate_scatter(counts_v, [best_idx, b_vec], ones16)
        return carry

    lax.fori_loop(0, NUM_BG, bg_body, 0)

    pltpu.sync_copy(counts_v, cnt_hbm.at[pl.ds(wid * NUM_HIVES, NUM_HIVES)])


_sc_vote = functools.partial(
    pl.kernel,
    out_type=jax.ShapeDtypeStruct((NUM_WORKERS * NUM_HIVES, B), jnp.float32),
    mesh=plsc.VectorSubcoreMesh(core_axis_name="c", subcore_axis_name="s"),
    compiler_params=pltpu.CompilerParams(needs_layout_passes=False),
    scratch_types=[
        pltpu.VMEM((A_PER_W, NUM_HIVES, B), jnp.float32),
        pltpu.VMEM((NUM_HIVES, B), jnp.float32),
    ],
)(_sc_vote_body)


def _tc_cost_body(cnt_ref, hv_ref, mv_ref, cost_ref, mf_ref):
    parts = cnt_ref[...].reshape(NUM_WORKERS, NUM_HIVES, B)
    counts = jnp.sum(parts, axis=0)
    mf = jnp.max(counts, axis=0) * (1.0 / NUM_AGENTS)
    val = jnp.sum(counts * hv_ref[:, 0, :], axis=0)
    term = val * (1.0 + jnp.exp(30.0 * (mf - 0.7))) * (1.0 / 100.0)

    mv = mv_ref[...]
    norms = jnp.sqrt(jnp.sum(mv * mv, axis=1))

    cost_ref[...] = jnp.reshape(jnp.sum(norms) - jnp.sum(term), (1, 1))
    mf_ref[...] = jnp.reshape(mf, (1, B))


def kernel(movements, utterances, votes, hive_values, locations):
    votes_t = jnp.transpose(votes, (1, 2, 0))
    parts = _sc_vote(votes_t)
    hv_t = jnp.transpose(hive_values, (1, 2, 0))
    mv_t = jnp.transpose(movements, (1, 2, 0))
    cost, mf = pl.pallas_call(
        _tc_cost_body,
        out_shape=(jax.ShapeDtypeStruct((1, 1), jnp.float32),
                   jax.ShapeDtypeStruct((1, B), jnp.float32)),
    )(parts, hv_t, mv_t)
    return (cost[0, 0], mf.reshape(B))


# speedup vs baseline: 2.1478x; 1.0026x over previous
import functools

import jax
import jax.numpy as jnp
from jax import lax
from jax.experimental import pallas as pl
from jax.experimental.pallas import tpu as pltpu
from jax.experimental.pallas import tpu_sc as plsc

B = 512
NUM_AGENTS = 64
NUM_HIVES = 16
NUM_ENTITIES = 80
LANES = 16
NUM_WORKERS = 32
A_PER_W = NUM_AGENTS // NUM_WORKERS
NUM_BG = B // LANES


def _sc_vote_body(votes_hbm, cnt_hbm, votes_v, counts_v):
    c = lax.axis_index("c")
    s = lax.axis_index("s")
    wid = s * 2 + c
    a0 = wid * A_PER_W

    pltpu.sync_copy(votes_hbm.at[pl.ds(a0, A_PER_W)], votes_v)

    lane = lax.iota(jnp.int32, LANES)
    zero16 = jnp.zeros((LANES,), jnp.float32)
    ones16 = jnp.ones((LANES,), jnp.float32)

    def zero_body(j, carry):
        for h in range(NUM_HIVES):
            counts_v[h, pl.ds(j * LANES, LANES)] = zero16
        return carry

    lax.fori_loop(0, B // LANES, zero_body, 0)

    def half_scan(a, bg, h_lo):
        best_val = votes_v[a, h_lo, pl.ds(bg * LANES, LANES)]
        best_idx = jnp.full((LANES,), h_lo, jnp.int32)
        for h in range(h_lo + 1, h_lo + 8):
            col = votes_v[a, h, pl.ds(bg * LANES, LANES)]
            m = col > best_val
            best_val = jnp.where(m, col, best_val)
            best_idx = jnp.where(m, jnp.full((LANES,), h, jnp.int32), best_idx)
        return best_val, best_idx

    def bg_body(bg, carry):
        b_vec = bg * LANES + lane
        for a in range(A_PER_W):
            val_lo, idx_lo = half_scan(a, bg, 0)
            val_hi, idx_hi = half_scan(a, bg, 8)
            m = val_hi > val_lo
            best_idx = jnp.where(m, idx_hi, idx_lo)
            plsc.addupdate_scatter(counts_v, [best_idx, b_vec], ones16)
        return carry

    lax.fori_loop(0, NUM_BG, bg_body, 0)

    pltpu.sync_copy(counts_v, cnt_hbm.at[pl.ds(wid * NUM_HIVES, NUM_HIVES)])


_sc_vote = functools.partial(
    pl.kernel,
    out_type=jax.ShapeDtypeStruct((NUM_WORKERS * NUM_HIVES, B), jnp.float32),
    mesh=plsc.VectorSubcoreMesh(core_axis_name="c", subcore_axis_name="s"),
    compiler_params=pltpu.CompilerParams(needs_layout_passes=False),
    scratch_types=[
        pltpu.VMEM((A_PER_W, NUM_HIVES, B), jnp.float32),
        pltpu.VMEM((NUM_HIVES, B), jnp.float32),
    ],
)(_sc_vote_body)


def _tc_mov_body(mv_ref, movsum_ref):
    mv = mv_ref[...]
    norms = jnp.sqrt(jnp.sum(mv * mv, axis=1))
    movsum_ref[...] = jnp.reshape(jnp.sum(norms), (1, 1))


def _tc_cost_body(cnt_ref, hv_ref, movsum_ref, cost_ref, mf_ref):
    parts = cnt_ref[...].reshape(NUM_WORKERS, NUM_HIVES, B)
    counts = jnp.sum(parts, axis=0)
    mf = jnp.max(counts, axis=0) * (1.0 / NUM_AGENTS)
    val = jnp.sum(counts * hv_ref[:, 0, :], axis=0)
    term = val * (1.0 + jnp.exp(30.0 * (mf - 0.7))) * (1.0 / 100.0)

    cost_ref[...] = jnp.reshape(movsum_ref[0, 0] - jnp.sum(term), (1, 1))
    mf_ref[...] = jnp.reshape(mf, (1, B))


def kernel(movements, utterances, votes, hive_values, locations):
    votes_t = jnp.transpose(votes, (1, 2, 0))
    parts = _sc_vote(votes_t)
    hv_t = jnp.transpose(hive_values, (1, 2, 0))
    mv_t = jnp.transpose(movements, (1, 2, 0))
    movsum = pl.pallas_call(
        _tc_mov_body,
        out_shape=jax.ShapeDtypeStruct((1, 1), jnp.float32),
    )(mv_t)
    cost, mf = pl.pallas_call(
        _tc_cost_body,
        out_shape=(jax.ShapeDtypeStruct((1, 1), jnp.float32),
                   jax.ShapeDtypeStruct((1, B), jnp.float32)),
    )(parts, hv_t, movsum)
    return (cost[0, 0], mf.reshape(B))


# speedup vs baseline: 2.1859x; 1.0177x over previous
import functools

import jax
import jax.numpy as jnp
from jax import lax
from jax.experimental import pallas as pl
from jax.experimental.pallas import tpu as pltpu
from jax.experimental.pallas import tpu_sc as plsc

B = 512
NUM_AGENTS = 64
NUM_HIVES = 16
NUM_ENTITIES = 80
LANES = 16
NUM_WORKERS = 32
A_PER_W = NUM_AGENTS // NUM_WORKERS
NUM_BG = B // LANES


def _sc_vote_body(votes_hbm, cnt_hbm, votes_v, counts_v, sem0, sem1):
    c = lax.axis_index("c")
    s = lax.axis_index("s")
    wid = s * 2 + c
    a0 = wid * A_PER_W

    cp0 = pltpu.make_async_copy(votes_hbm.at[a0], votes_v.at[0], sem0)
    cp1 = pltpu.make_async_copy(votes_hbm.at[a0 + 1], votes_v.at[1], sem1)
    cp0.start()
    cp1.start()

    lane = lax.iota(jnp.int32, LANES)
    zero16 = jnp.zeros((LANES,), jnp.float32)
    ones16 = jnp.ones((LANES,), jnp.float32)

    def zero_body(j, carry):
        for h in range(NUM_HIVES):
            counts_v[h, pl.ds(j * LANES, LANES)] = zero16
        return carry

    lax.fori_loop(0, B // LANES, zero_body, 0)
    cp0.wait()
    cp1.wait()

    def half_scan(a, bg, h_lo):
        best_val = votes_v[a, h_lo, pl.ds(bg * LANES, LANES)]
        best_idx = jnp.full((LANES,), h_lo, jnp.int32)
        for h in range(h_lo + 1, h_lo + 8):
            col = votes_v[a, h, pl.ds(bg * LANES, LANES)]
            m = col > best_val
            best_val = jnp.where(m, col, best_val)
            best_idx = jnp.where(m, jnp.full((LANES,), h, jnp.int32), best_idx)
        return best_val, best_idx

    def bg_body(i, carry):
        for u in range(2):
            bg = i * 2 + u
            b_vec = bg * LANES + lane
            for a in range(A_PER_W):
                val_lo, idx_lo = half_scan(a, bg, 0)
                val_hi, idx_hi = half_scan(a, bg, 8)
                m = val_hi > val_lo
                best_idx = jnp.where(m, idx_hi, idx_lo)
                plsc.addupdate_scatter(counts_v, [best_idx, b_vec], ones16)
        return carry

    lax.fori_loop(0, NUM_BG // 2, bg_body, 0)

    pltpu.sync_copy(counts_v, cnt_hbm.at[pl.ds(wid * NUM_HIVES, NUM_HIVES)])


_sc_vote = functools.partial(
    pl.kernel,
    out_type=jax.ShapeDtypeStruct((NUM_WORKERS * NUM_HIVES, B), jnp.float32),
    mesh=plsc.VectorSubcoreMesh(core_axis_name="c", subcore_axis_name="s"),
    compiler_params=pltpu.CompilerParams(needs_layout_passes=False),
    scratch_types=[
        pltpu.VMEM((A_PER_W, NUM_HIVES, B), jnp.float32),
        pltpu.VMEM((NUM_HIVES, B), jnp.float32),
        pltpu.SemaphoreType.DMA,
        pltpu.SemaphoreType.DMA,
    ],
)(_sc_vote_body)


def _tc_mov_body(mv_ref, movsum_ref):
    mv = mv_ref[...]
    norms = jnp.sqrt(jnp.sum(mv * mv, axis=1))
    movsum_ref[...] = jnp.reshape(jnp.sum(norms), (1, 1))


def _tc_cost_body(cnt_ref, hv_ref, movsum_ref, cost_ref, mf_ref):
    parts = cnt_ref[...].reshape(NUM_WORKERS, NUM_HIVES, B)
    counts = jnp.sum(parts, axis=0)
    mf = jnp.max(counts, axis=0) * (1.0 / NUM_AGENTS)
    val = jnp.sum(counts * hv_ref[:, 0, :], axis=0)
    term = val * (1.0 + jnp.exp(30.0 * (mf - 0.7))) * (1.0 / 100.0)

    cost_ref[...] = jnp.reshape(movsum_ref[0, 0] - jnp.sum(term), (1, 1))
    mf_ref[...] = jnp.reshape(mf, (1, B))


def kernel(movements, utterances, votes, hive_values, locations):
    votes_t = jnp.transpose(votes, (1, 2, 0))
    parts = _sc_vote(votes_t)
    hv_t = jnp.transpose(hive_values, (1, 2, 0))
    mv_t = jnp.transpose(movements, (1, 2, 0))
    movsum = pl.pallas_call(
        _tc_mov_body,
        out_shape=jax.ShapeDtypeStruct((1, 1), jnp.float32),
    )(mv_t)
    cost, mf = pl.pallas_call(
        _tc_cost_body,
        out_shape=(jax.ShapeDtypeStruct((1, 1), jnp.float32),
                   jax.ShapeDtypeStruct((1, B), jnp.float32)),
    )(parts, hv_t, movsum)
    return (cost[0, 0], mf.reshape(B))
